# baseline (speedup 1.0000x reference)
import functools

import jax
import jax.numpy as jnp
from jax import lax
from jax.experimental import pallas as pl
from jax.experimental.pallas import tpu as pltpu
from jax.experimental.pallas import tpu_sc as plsc

NUM_ITEMS = 1000000
D = 32
PACK = 128 // D
B = 16384
NC = 2
NS = 16
NW = NC * NS
B_PER_W = B // NW
CHUNK = 128
N_CHUNKS = B_PER_W // CHUNK
BLOCKS = B_PER_W // 16


def _lane_perm(t, p):
    dnums = lax.GatherDimensionNumbers(
        offset_dims=(), collapsed_slice_dims=(0,), start_index_map=(0,))
    return lax.gather(t, p[:, None], dnums, slice_sizes=(1,),
                      mode=lax.GatherScatterMode.PROMISE_IN_BOUNDS)


def _make_sc_kernel():
    mesh = plsc.VectorSubcoreMesh(core_axis_name="c", subcore_axis_name="s")

    @functools.partial(
        pl.kernel,
        mesh=mesh,
        out_type=jax.ShapeDtypeStruct((B,), jnp.float32),
        scratch_types=[
            pltpu.VMEM((N_CHUNKS, CHUNK), jnp.int32),
            pltpu.VMEM((B_PER_W,), jnp.int32),
            pltpu.VMEM((B_PER_W, 128), jnp.float32),
            pltpu.VMEM((D,), jnp.float32),
            pltpu.VMEM((B_PER_W,), jnp.float32),
            pltpu.SemaphoreType.DMA,
        ],
    )
    def fcf_kernel(pk_hbm, off_hbm, table_hbm, u_hbm, out_hbm,
                   pk_v, off_v, rows_v, u_v, out_v, sem):
        wid = lax.axis_index("s") * NC + lax.axis_index("c")
        base = wid * B_PER_W

        pltpu.sync_copy(pk_hbm.at[wid], pk_v)
        pltpu.sync_copy(off_hbm.at[wid], off_v)
        pltpu.sync_copy(u_hbm, u_v)

        copies = []
        for j in range(N_CHUNKS):
            copies.append(pltpu.async_copy(
                table_hbm.at[pk_v.at[j]],
                rows_v.at[pl.ds(j * CHUNK, CHUNK)],
                sem,
            ))
        for c in copies:
            c.wait()

        u_lo = u_v[pl.ds(0, 16)]
        u_hi = u_v[pl.ds(16, 16)]
        lane = lax.iota(jnp.int32, 16)
        perms = [lane ^ jnp.int32(s) for s in (1, 2, 4, 8)]

        def block_body(g, carry):
            ov = off_v[pl.ds(g * 16, 16)]
            acc = jnp.zeros((16,), jnp.float32)
            for i in range(16):
                k = g * 16 + i
                o = ov[i]
                t = (rows_v[k, pl.ds(o, 16)] * u_lo
                     + rows_v[k, pl.ds(o + 16, 16)] * u_hi)
                for p in perms:
                    t = t + _lane_perm(t, p)
                acc = jnp.where(lane == i, t, acc)
            out_v[pl.ds(g * 16, 16)] = 1.0 / (1.0 + jnp.exp(-acc))
            return carry

        lax.fori_loop(0, BLOCKS, block_body, jnp.int32(0))

        pltpu.sync_copy(out_v, out_hbm.at[pl.ds(base, B_PER_W)])

    return fcf_kernel


_fcf_sc = _make_sc_kernel()


def kernel(item_indices, item_table, user_embedding):
    idx = item_indices.astype(jnp.int32)
    pk = (idx // PACK).reshape(NW, N_CHUNKS, CHUNK)
    off = ((idx % PACK) * D).reshape(NW, B_PER_W)
    table128 = item_table.reshape(NUM_ITEMS // PACK, 128)
    u = user_embedding.reshape(D)
    return _fcf_sc(pk, off, table128, u)


# speedup vs baseline: 2.4650x; 2.4650x over previous
import functools

import jax
import jax.numpy as jnp
from jax import lax
from jax.experimental import pallas as pl
from jax.experimental.pallas import tpu as pltpu
from jax.experimental.pallas import tpu_sc as plsc

NUM_ITEMS = 1000000
D = 32
TROWS = 8
NT = NUM_ITEMS // TROWS
B = 16384
NC = 2
NS = 16
NW = NC * NS
B_PER_W = B // NW
ROUND = 64
N_ROUNDS = B_PER_W // ROUND
BLOCKS = ROUND // 16


def _lane_perm(t, p):
    dnums = lax.GatherDimensionNumbers(
        offset_dims=(), collapsed_slice_dims=(0,), start_index_map=(0,))
    return lax.gather(t, p[:, None], dnums, slice_sizes=(1,),
                      mode=lax.GatherScatterMode.PROMISE_IN_BOUNDS)


def _make_sc_kernel():
    mesh = plsc.VectorSubcoreMesh(core_axis_name="c", subcore_axis_name="s")

    @functools.partial(
        pl.kernel,
        mesh=mesh,
        out_type=jax.ShapeDtypeStruct((B,), jnp.float32),
        scratch_types=[
            pltpu.VMEM((N_ROUNDS, ROUND), jnp.int32),
            pltpu.VMEM((B_PER_W,), jnp.int32),
            pltpu.VMEM((ROUND, TROWS, D), jnp.float32),
            pltpu.VMEM((D,), jnp.float32),
            pltpu.VMEM((B_PER_W,), jnp.float32),
            pltpu.SemaphoreType.DMA,
        ],
    )
    def fcf_kernel(pk_hbm, sub_hbm, table_hbm, u_hbm, out_hbm,
                   pk_v, sub_v, tiles_v, u_v, out_v, sem):
        wid = lax.axis_index("s") * NC + lax.axis_index("c")
        base = wid * B_PER_W

        pltpu.sync_copy(pk_hbm.at[wid], pk_v)
        pltpu.sync_copy(sub_hbm.at[wid], sub_v)
        pltpu.sync_copy(u_hbm, u_v)

        u_lo = u_v[pl.ds(0, 16)]
        u_hi = u_v[pl.ds(16, 16)]
        lane = lax.iota(jnp.int32, 16)
        perms = [lane ^ jnp.int32(s) for s in (1, 2, 4, 8)]

        def round_body(r, carry):
            for q in range(ROUND // 16):
                pkv = pk_v[r, pl.ds(q * 16, 16)]
                for j in range(16):
                    pltpu.async_copy(
                        table_hbm.at[pkv[j]], tiles_v.at[q * 16 + j], sem)
            pltpu.make_async_copy(
                table_hbm.at[pl.ds(0, ROUND)], tiles_v, sem).wait()
            for g in range(BLOCKS):
                sv = sub_v[pl.ds(r * ROUND + g * 16, 16)]
                acc = jnp.zeros((16,), jnp.float32)
                for i in range(16):
                    k = g * 16 + i
                    s = sv[i]
                    t = (tiles_v[k, s, pl.ds(0, 16)] * u_lo
                         + tiles_v[k, s, pl.ds(16, 16)] * u_hi)
                    for p in perms:
                        t = t + _lane_perm(t, p)
                    acc = jnp.where(lane == i, t, acc)
                out_v[pl.ds(r * ROUND + g * 16, 16)] = (
                    1.0 / (1.0 + jnp.exp(-acc)))
            return carry

        lax.fori_loop(0, N_ROUNDS, round_body, jnp.int32(0))

        pltpu.sync_copy(out_v, out_hbm.at[pl.ds(base, B_PER_W)])

    return fcf_kernel


_fcf_sc = _make_sc_kernel()


def kernel(item_indices, item_table, user_embedding):
    idx = item_indices.astype(jnp.int32)
    pk = (idx // TROWS).reshape(NW, N_ROUNDS, ROUND)
    sub = (idx % TROWS).reshape(NW, B_PER_W)
    table3 = item_table.reshape(NT, TROWS, D)
    u = user_embedding.reshape(D)
    return _fcf_sc(pk, sub, table3, u)
